# baseline (speedup 1.0000x reference)
import functools
import jax
import jax.numpy as jnp
from jax import lax
from jax.experimental import pallas as pl
from jax.experimental.pallas import tpu as pltpu
from jax.experimental.pallas import tpu_sc as plsc

N_NODE = 100000
D = 100
B = 128
L = 20
N = B * L
K = 12
SB = 16
RB = SB * L
GRID_BC = B // SB
VB = 2048
GRID_D = (N_NODE + VB - 1) // VB


def _leaky(v, s):
    return jnp.where(v >= 0, v, s * v)


NIDX = N + N * K
NW = 32
IPW = NIDX // NW
_GCH = (128,) * 8 + (16,)


def _sc_gather_body(idx_hbm, emb_hbm, out_hbm, idx_v, rows_v, sem):
    wid = lax.axis_index("s") * 2 + lax.axis_index("c")
    base = wid * IPW
    pltpu.sync_copy(idx_hbm.at[pl.ds(base, IPW)], idx_v)
    copies = []
    off = 0
    for ch in _GCH:
        copies.append(pltpu.async_copy(
            emb_hbm.at[idx_v.at[pl.ds(off, ch)]],
            rows_v.at[pl.ds(off, ch)], sem))
        off += ch
    for c in copies:
        c.wait()
    pltpu.sync_copy(rows_v, out_hbm.at[pl.ds(base, IPW)])


def _run_sc_gather(idx, emb):
    f = pl.kernel(
        _sc_gather_body,
        out_type=jax.ShapeDtypeStruct((NIDX, D), jnp.float32),
        mesh=plsc.VectorSubcoreMesh(core_axis_name="c", subcore_axis_name="s"),
        scratch_types=[
            pltpu.VMEM((IPW,), jnp.int32),
            pltpu.VMEM((IPW, D), jnp.float32),
            pltpu.SemaphoreType.DMA,
        ],
        compiler_params=pltpu.CompilerParams(use_tc_tiling_on_sc=False),
    )
    return f(idx, emb)


E = 4 * N + N
EPT = E // 16
T2 = N * N
HALF = T2 // 2
_SELROWS = (2 * EPT + 127) // 128 + 1
_ZB = 8192
_ZPT = HALF // 16


def _sc_mask_body(e0_hbm, e1_hbm, out_hbm, u_v, v_v, sel_v, ones_v, zer_v,
                  zsem, ssem):
    c = lax.axis_index("c")
    s = lax.axis_index("s")
    zero16 = jnp.zeros((16,), jnp.float32)

    def _fill(j, _):
        zer_v[pl.ds(j * 16, 16)] = zero16
        return 0

    lax.fori_loop(0, _ZB // 16, _fill, 0)
    for j in range(128 // 16):
        ones_v[pl.ds(j * 16, 16)] = zero16 + 1.0

    zoff = c * HALF + s * _ZPT
    zcopies = [
        pltpu.async_copy(zer_v, out_hbm.at[pl.ds(zoff + k * _ZB, _ZB)], zsem)
        for k in range(_ZPT // _ZB)
    ]

    pltpu.sync_copy(e0_hbm.at[pl.ds(s * EPT, EPT)], u_v)
    pltpu.sync_copy(e1_hbm.at[pl.ds(s * EPT, EPT)], v_v)
    lo = c * HALF
    safe = c * (1280 * (N + 1))
    for t in range(EPT // 16):
        ul = u_v[pl.ds(t * 16, 16)]
        vl = v_v[pl.ds(t * 16, 16)]
        f0 = ul * N + vl
        f1 = vl * N + ul
        s0 = jnp.where((f0 >= lo) & (f0 < lo + HALF), f0, safe)
        s1 = jnp.where((f1 >= lo) & (f1 < lo + HALF), f1, safe)
        p0 = t * 16
        p1 = 2 * EPT + t * 16
        sel_v[p0 // 128, pl.ds(p0 % 128, 16)] = s0
        sel_v[p1 // 128, pl.ds(p1 % 128, 16)] = s1
    for t in range(2 * EPT // 16, _SELROWS * 8):
        p = t * 16
        sel_v[p // 128, pl.ds(p % 128, 16)] = jnp.zeros((16,), jnp.int32) + safe

    for zc in zcopies:
        zc.wait()
    plsc.subcore_barrier()

    scopies = [pltpu.async_copy(ones_v, out_hbm.at[sel_v.at[j]], ssem)
               for j in range(_SELROWS)]
    for sc_ in scopies:
        sc_.wait()


def _run_sc_mask(e0, e1):
    f = pl.kernel(
        _sc_mask_body,
        out_type=jax.ShapeDtypeStruct((T2,), jnp.float32),
        mesh=plsc.VectorSubcoreMesh(core_axis_name="c", subcore_axis_name="s"),
        scratch_types=[
            pltpu.VMEM((EPT,), jnp.int32),
            pltpu.VMEM((EPT,), jnp.int32),
            pltpu.VMEM((_SELROWS, 128), jnp.int32),
            pltpu.VMEM((128,), jnp.float32),
            pltpu.VMEM((_ZB,), jnp.float32),
            pltpu.SemaphoreType.DMA,
            pltpu.SemaphoreType.DMA,
        ],
    )
    return f(e0, e1)


def _global_body(x_ref, nw_ref, nb_ref, hid_ref, w1_ref, b1_ref, q1_ref,
                 w2_ref, b2_ref, out_ref):
    xb = x_ref[...].astype(jnp.float32)
    s_mean = jnp.mean(xb, axis=1)
    w_soft = jax.nn.softmax(nw_ref[...], axis=-1)
    nb = nb_ref[...].reshape(SB, L, K, D)
    sh = s_mean[:, None, None, None] * nb
    feat = jnp.concatenate([sh, w_soft[..., None]], axis=-1)
    a = jnp.dot(feat.reshape(SB * L * K, D + 1), w1_ref[...],
                preferred_element_type=jnp.float32) + b1_ref[...]
    a = _leaky(a, 0.01)
    a = jnp.dot(a, q1_ref[...], preferred_element_type=jnp.float32)
    a = a.reshape(SB, L, K)
    alpha = jax.nn.softmax(a, axis=-1)
    h_n = jnp.sum(alpha[..., None] * nb, axis=2)
    hcat = jnp.concatenate([hid_ref[...], h_n.reshape(RB, D)], axis=1)
    hg = jnp.dot(hcat, w2_ref[...], preferred_element_type=jnp.float32)
    out_ref[...] = jnp.maximum(hg + b2_ref[...], 0.0)


def _run_global(x2d, neighbor_w, nbflat, hidden, W1, b1, q1, W2, b2):
    full = lambda shp: pl.BlockSpec(shp, lambda i: (0,) * len(shp))
    return pl.pallas_call(
        _global_body,
        grid=(GRID_BC,),
        in_specs=[
            pl.BlockSpec((SB, L), lambda i: (i, 0)),
            pl.BlockSpec((SB, L, K), lambda i: (i, 0, 0)),
            pl.BlockSpec((RB * K, D), lambda i: (i, 0)),
            pl.BlockSpec((RB, D), lambda i: (i, 0)),
            full((D + 1, D + 1)),
            full((1, D + 1)),
            full((D + 1, 1)),
            full((2 * D, D)),
            full((1, D)),
        ],
        out_specs=pl.BlockSpec((RB, D), lambda i: (i, 0)),
        out_shape=jax.ShapeDtypeStruct((N, D), jnp.float32),
    )(x2d, neighbor_w, nbflat, hidden, W1, b1.reshape(1, D + 1),
      q1.reshape(D + 1, 1), W2, b2.reshape(1, D))


def _local_body(hid_ref, hblk_ref, hg_ref, mask_ref, av_ref, pos_ref,
                w3_ref, b3_ref, w4_ref, w5_ref, b5_ref, q2_ref, out_ref):
    hid = hid_ref[...]
    hblk = hblk_ref[...]
    q = hblk * av_ref[...]
    pre = lax.dot_general(q, hid, (((1,), (1,)), ((), ())),
                          preferred_element_type=jnp.float32)
    e = _leaky(pre, 0.2)
    m = mask_ref[...] > 0.0
    e = jnp.where(m, e, -1e30)
    emax = jnp.max(e, axis=1, keepdims=True)
    ex = jnp.exp(e - emax)
    ex = jnp.where(m, ex, 0.0)
    att = ex / jnp.sum(ex, axis=1, keepdims=True)
    h = jnp.dot(att, hid, preferred_element_type=jnp.float32) + hg_ref[...]
    pos_rep = jnp.broadcast_to(pos_ref[...][:, None, :], (SB, L, D))
    pos_rep = pos_rep.reshape(RB, D)
    z = jnp.tanh(
        jnp.dot(jnp.concatenate([h, pos_rep], axis=1), w3_ref[...],
                preferred_element_type=jnp.float32) + b3_ref[...])
    s_sess = jnp.mean(h.reshape(SB, L, D), axis=1)
    s_rep = jnp.broadcast_to(s_sess[:, None, :], (SB, L, D)).reshape(RB, D)
    gate = jax.nn.sigmoid(
        jnp.dot(z, w4_ref[...], preferred_element_type=jnp.float32)
        + jnp.dot(s_rep, w5_ref[...], preferred_element_type=jnp.float32)
        + b5_ref[...])
    beta = jnp.dot(gate, q2_ref[...], preferred_element_type=jnp.float32)
    out_ref[...] = jnp.sum((beta * h).reshape(SB, L, D), axis=1)


def _run_local(hidden, h_global, mask, a_vec, pos, W3, b3, W4, W5, b5, q2):
    full = lambda shp: pl.BlockSpec(shp, lambda i: (0,) * len(shp))
    return pl.pallas_call(
        _local_body,
        grid=(GRID_BC,),
        in_specs=[
            full((N, D)),
            pl.BlockSpec((RB, D), lambda i: (i, 0)),
            pl.BlockSpec((RB, D), lambda i: (i, 0)),
            pl.BlockSpec((RB, N), lambda i: (i, 0)),
            full((1, D)),
            pl.BlockSpec((SB, D), lambda i: (i, 0)),
            full((2 * D, D)),
            full((1, D)),
            full((D, D)),
            full((D, D)),
            full((1, D)),
            full((D, 1)),
        ],
        out_specs=pl.BlockSpec((SB, D), lambda i: (i, 0)),
        out_shape=jax.ShapeDtypeStruct((B, D), jnp.float32),
    )(hidden, hidden, h_global, mask, a_vec.reshape(1, D), pos[:B],
      W3, b3.reshape(1, D), W4, W5, b5.reshape(1, D), q2.reshape(D, 1))


def _scores_body(s_ref, emb_ref, out_ref):
    out_ref[...] = lax.dot_general(
        s_ref[...], emb_ref[...], (((1,), (1,)), ((), ())),
        preferred_element_type=jnp.float32)


def _run_scores(S, emb):
    return pl.pallas_call(
        _scores_body,
        grid=(GRID_D,),
        in_specs=[
            pl.BlockSpec((B, D), lambda i: (0, 0)),
            pl.BlockSpec((VB, D), lambda i: (i, 0)),
        ],
        out_specs=pl.BlockSpec((B, VB), lambda i: (0, i)),
        out_shape=jax.ShapeDtypeStruct((B, N_NODE), jnp.float32),
    )(S, emb)


def kernel(x, edge_index, neighbor_ids, neighbor_w, emb, pos, W1, b1, q1,
           W2, b2, W3, b3, q2, W4, W5, b5, a_vec):
    idx = jnp.concatenate([x.astype(jnp.int32),
                           neighbor_ids.reshape(-1).astype(jnp.int32)])
    rows = _run_sc_gather(idx, emb)
    hidden = rows[:N]
    nbflat = rows[N:]
    ei = edge_index.astype(jnp.int32)
    mask = _run_sc_mask(ei[0], ei[1]).reshape(N, N)

    h_global = _run_global(x.reshape(B, L), neighbor_w, nbflat, hidden,
                           W1, b1, q1, W2, b2)
    S = _run_local(hidden, h_global, mask, a_vec, pos, W3, b3, W4, W5, b5, q2)
    return _run_scores(S, emb)
